# baseline (speedup 1.0000x reference)
import functools
import jax
import jax.numpy as jnp
from jax import lax
from jax.experimental import pallas as pl
from jax.experimental.pallas import tpu as pltpu
from jax.experimental.pallas import tpu_sc as plsc

_VOCAB = 1000000
_D = 32
_DO = 16
_B = 16384
_SEQ = 200

_NW = 32
_BAGS_W = _B // _NW
_CB = 16
_CHUNKS = _BAGS_W // _CB
_GRP = 128
_NGRP = _CB * _SEQ // _GRP
_IDX_W = _BAGS_W * _SEQ // _GRP


_NTILE = 7813
_VPAD = _NTILE * 128


def _transform(table_t, fc_w):
  blkc = 7808

  def body(tt_ref, w_ref, out_ref):
    out_ref[...] = lax.dot_general(
        w_ref[...], tt_ref[...],
        dimension_numbers=(((1,), (0,)), ((), ())),
        preferred_element_type=jnp.float32) * (1.0 / _SEQ)

  return pl.pallas_call(
      body,
      grid=((_VPAD + blkc - 1) // blkc,),
      in_specs=[
          pl.BlockSpec((_D, blkc), lambda i: (0, i)),
          pl.BlockSpec((_DO, _D), lambda i: (0, 0)),
      ],
      out_specs=pl.BlockSpec((_DO, blkc), lambda i: (0, i)),
      out_shape=jax.ShapeDtypeStruct((_DO, _VPAD), jnp.float32),
  )(table_t, fc_w)


_NCH = _NTILE // 8


@functools.partial(
    pl.kernel,
    mesh=plsc.VectorSubcoreMesh(core_axis_name="c", subcore_axis_name="s"),
    out_type=jax.ShapeDtypeStruct((_VPAD, _DO), jnp.float32),
    compiler_params=pltpu.CompilerParams(use_tc_tiling_on_sc=False,
                                         needs_layout_passes=False),
    scratch_types=[
        pltpu.VMEM((_DO, 8, 128), jnp.float32),
        pltpu.VMEM((_DO, 8, 128), jnp.float32),
        pltpu.VMEM((1024, _DO), jnp.float32),
        pltpu.SemaphoreType.DMA,
        pltpu.SemaphoreType.DMA,
    ],
)
def _untile(t4_hbm, out_hbm, stg_a, stg_b, rows_v, sem_a, sem_b):
  w = lax.axis_index("s") * 2 + lax.axis_index("c")
  iota = lax.iota(jnp.int32, 16)
  rsel = iota % 8

  def fetch(ch, stg, sem):
    pltpu.async_copy(t4_hbm.at[0, pl.ds(ch * 8, 8), :, :],
                     stg.at[pl.ds(0, 8), :, :], sem)
    pltpu.async_copy(t4_hbm.at[1, pl.ds(ch * 8, 8), :, :],
                     stg.at[pl.ds(8, 8), :, :], sem)

  def drain(ch, stg, sem):
    pltpu.make_async_copy(t4_hbm.at[0, pl.ds(ch * 8, 8), :, :],
                          stg.at[pl.ds(0, 8), :, :], sem).wait()
    pltpu.make_async_copy(t4_hbm.at[1, pl.ds(ch * 8, 8), :, :],
                          stg.at[pl.ds(8, 8), :, :], sem).wait()

  def xpose(ch, stg, ntiles):
    for t in range(ntiles):
      xsel = (iota // 8) * 8 + t

      @plsc.parallel_loop(0, 128, step=1, unroll=16)
      def _(j):
        rows_v[t * 128 + j, :] = plsc.load_gather(
            stg, [xsel, rsel, iota * 0 + j])

    pltpu.sync_copy(rows_v.at[pl.ds(0, ntiles * 128), :],
                    out_hbm.at[pl.ds(ch * 1024, ntiles * 128), :])

  fetch(w, stg_a, sem_a)

  def pair(t, carry):
    s0, s1 = 2 * t, 2 * t + 1
    ch0, ch1 = w + 32 * s0, w + 32 * s1

    @pl.when(ch1 < _NCH)
    def _():
      fetch(ch1, stg_b, sem_b)

    @pl.when(ch0 < _NCH)
    def _():
      drain(ch0, stg_a, sem_a)
      xpose(ch0, stg_a, 8)

    ch2 = w + 32 * (s0 + 2)

    @pl.when(ch2 < _NCH)
    def _():
      fetch(ch2, stg_a, sem_a)

    @pl.when(ch1 < _NCH)
    def _():
      drain(ch1, stg_b, sem_b)
      xpose(ch1, stg_b, 8)

    return carry

  lax.fori_loop(0, (_NCH + 63) // 64 + 1, pair, 0)

  @pl.when(w < _NTILE % 8)
  def _():
    g = _NCH * 8 + w
    pltpu.sync_copy(t4_hbm.at[0, pl.ds(g, 1), :, :], stg_a.at[pl.ds(0, 1), :, :])
    pltpu.sync_copy(t4_hbm.at[1, pl.ds(g, 1), :, :], stg_a.at[pl.ds(8, 1), :, :])
    xsel = (iota // 8) * 8

    @plsc.parallel_loop(0, 128, step=1, unroll=16)
    def _(j):
      rows_v[j, :] = plsc.load_gather(stg_a, [xsel, rsel, iota * 0 + j])
    pltpu.sync_copy(rows_v.at[pl.ds(0, 128), :],
                    out_hbm.at[pl.ds(g * 128, 128), :])


@functools.partial(
    pl.kernel,
    mesh=plsc.VectorSubcoreMesh(core_axis_name="c", subcore_axis_name="s"),
    out_type=jax.ShapeDtypeStruct((_B, _DO), jnp.float32),
    compiler_params=pltpu.CompilerParams(use_tc_tiling_on_sc=False),
    scratch_types=[
        pltpu.VMEM((_CB * _SEQ,), jnp.int32),
        pltpu.VMEM((_CB * _SEQ,), jnp.int32),
        pltpu.VMEM((_CB * _SEQ, _DO), jnp.float32),
        pltpu.VMEM((_CB * _SEQ, _DO), jnp.float32),
        pltpu.VMEM((_CB, _DO), jnp.float32),
        pltpu.VMEM((_DO,), jnp.float32),
        pltpu.SemaphoreType.DMA,
        pltpu.SemaphoreType.DMA,
    ],
)
def _bagsum(ids_hbm, t2_hbm, b_hbm, out_hbm, idx_a, idx_b, rows_a, rows_b,
            stg_v, bias_v, sem_a, sem_b):
  w = lax.axis_index("s") * 2 + lax.axis_index("c")
  pltpu.sync_copy(b_hbm, bias_v)
  bias = bias_v[...]

  def fetch(c, idx_v, rows_v, sem):
    pltpu.sync_copy(
        ids_hbm.at[pl.ds(w * _BAGS_W * _SEQ + c * _CB * _SEQ, _CB * _SEQ)],
        idx_v)
    for g in range(_NGRP):
      pltpu.async_copy(t2_hbm.at[idx_v.at[pl.ds(g * _GRP, _GRP)]],
                       rows_v.at[pl.ds(g * _GRP, _GRP), :], sem)

  def drain(idx_v, rows_v, sem):
    for g in range(_NGRP):
      pltpu.make_async_copy(t2_hbm.at[idx_v.at[pl.ds(g * _GRP, _GRP)]],
                            rows_v.at[pl.ds(g * _GRP, _GRP), :], sem).wait()

  def reduce(c, rows_v):
    def red(i, accs):
      return tuple(accs[b] + rows_v[b * _SEQ + i, :] for b in range(_CB))

    accs = lax.fori_loop(0, _SEQ, red, tuple(bias for _ in range(_CB)))
    for b in range(_CB):
      stg_v[b, :] = accs[b]
    pltpu.sync_copy(stg_v, out_hbm.at[pl.ds(w * _BAGS_W + c * _CB, _CB), :])

  fetch(0, idx_a, rows_a, sem_a)

  def pair(t, carry):
    c = 2 * t
    fetch(c + 1, idx_b, rows_b, sem_b)
    drain(idx_a, rows_a, sem_a)
    reduce(c, rows_a)
    fetch(jnp.minimum(c + 2, _CHUNKS - 1), idx_a, rows_a, sem_a)
    drain(idx_b, rows_b, sem_b)
    reduce(c + 1, rows_b)
    return carry

  lax.fori_loop(0, _CHUNKS // 2, pair, 0)
  drain(idx_a, rows_a, sem_a)


def kernel(input_ids, table, fc_w, fc_b):
  t2t = _transform(lax.optimization_barrier(table.T), fc_w)
  t4 = t2t.reshape(2, 8, _NTILE, 128).transpose(0, 2, 1, 3)
  t2 = _untile(t4)
  ids_r = input_ids.astype(jnp.int32).reshape(_B * _SEQ)
  return _bagsum(ids_r, t2, fc_b)


# speedup vs baseline: 1.0392x; 1.0392x over previous
import functools
import jax
import jax.numpy as jnp
from jax import lax
from jax.experimental import pallas as pl
from jax.experimental.pallas import tpu as pltpu
from jax.experimental.pallas import tpu_sc as plsc

_VOCAB = 1000000
_D = 32
_DO = 16
_B = 16384
_SEQ = 200

_NW = 32
_BAGS_W = _B // _NW
_CB = 16
_CHUNKS = _BAGS_W // _CB
_GRP = 128
_NGRP = _CB * _SEQ // _GRP
_IDX_W = _BAGS_W * _SEQ // _GRP


_NTILE = 7813
_VPAD = _NTILE * 128


def _transform(table_t, fc_w):
  blkc = 7808

  def body(tt_ref, w_ref, out_ref):
    out_ref[...] = lax.dot_general(
        w_ref[...], tt_ref[...],
        dimension_numbers=(((1,), (0,)), ((), ())),
        preferred_element_type=jnp.float32) * (1.0 / _SEQ)

  return pl.pallas_call(
      body,
      grid=((_VPAD + blkc - 1) // blkc,),
      in_specs=[
          pl.BlockSpec((_D, blkc), lambda i: (0, i)),
          pl.BlockSpec((_DO, _D), lambda i: (0, 0)),
      ],
      out_specs=pl.BlockSpec((_DO, blkc), lambda i: (0, i)),
      out_shape=jax.ShapeDtypeStruct((_DO, _VPAD), jnp.float32),
  )(table_t, fc_w)


_NCH = _NTILE // 8


@functools.partial(
    pl.kernel,
    mesh=plsc.VectorSubcoreMesh(core_axis_name="c", subcore_axis_name="s"),
    out_type=jax.ShapeDtypeStruct((_VPAD, _DO), jnp.float32),
    compiler_params=pltpu.CompilerParams(use_tc_tiling_on_sc=False,
                                         needs_layout_passes=False),
    scratch_types=[
        pltpu.VMEM((_DO, 8, 128), jnp.float32),
        pltpu.VMEM((_DO, 8, 128), jnp.float32),
        pltpu.VMEM((1024, _DO), jnp.float32),
        pltpu.VMEM((1024, _DO), jnp.float32),
        pltpu.SemaphoreType.DMA,
        pltpu.SemaphoreType.DMA,
        pltpu.SemaphoreType.DMA,
        pltpu.SemaphoreType.DMA,
    ],
)
def _untile(t4_hbm, out_hbm, stg_a, stg_b, rows_va, rows_vb, sem_a, sem_b,
            semo_a, semo_b):
  w = lax.axis_index("s") * 2 + lax.axis_index("c")
  iota = lax.iota(jnp.int32, 16)
  rsel = iota % 8

  def fetch(ch, stg, sem):
    pltpu.async_copy(t4_hbm.at[0, pl.ds(ch * 8, 8), :, :],
                     stg.at[pl.ds(0, 8), :, :], sem)
    pltpu.async_copy(t4_hbm.at[1, pl.ds(ch * 8, 8), :, :],
                     stg.at[pl.ds(8, 8), :, :], sem)

  def drain(ch, stg, sem):
    pltpu.make_async_copy(t4_hbm.at[0, pl.ds(ch * 8, 8), :, :],
                          stg.at[pl.ds(0, 8), :, :], sem).wait()
    pltpu.make_async_copy(t4_hbm.at[1, pl.ds(ch * 8, 8), :, :],
                          stg.at[pl.ds(8, 8), :, :], sem).wait()

  def xpose(stg, rows_v):
    for t in range(8):
      xsel = (iota // 8) * 8 + t

      @plsc.parallel_loop(0, 128, step=1, unroll=16)
      def _(j):
        rows_v[t * 128 + j, :] = plsc.load_gather(
            stg, [xsel, rsel, iota * 0 + j])

  def flush(ch, rows_v, semo):
    pltpu.async_copy(rows_v, out_hbm.at[pl.ds(ch * 1024, 1024), :], semo)

  def drain_flush(ch, rows_v, semo):
    pltpu.make_async_copy(rows_v, out_hbm.at[pl.ds(ch * 1024, 1024), :],
                          semo).wait()

  fetch(w, stg_a, sem_a)

  def pair(t, carry):
    s0, s1 = 2 * t, 2 * t + 1
    ch0, ch1 = w + 32 * s0, w + 32 * s1

    @pl.when(ch1 < _NCH)
    def _():
      fetch(ch1, stg_b, sem_b)

    @pl.when(ch0 < _NCH)
    def _():
      drain(ch0, stg_a, sem_a)

      @pl.when(s0 >= 2)
      def _():
        drain_flush(ch0 - 64, rows_va, semo_a)

      xpose(stg_a, rows_va)
      flush(ch0, rows_va, semo_a)

    ch2 = w + 32 * (s0 + 2)

    @pl.when(ch2 < _NCH)
    def _():
      fetch(ch2, stg_a, sem_a)

    @pl.when(ch1 < _NCH)
    def _():
      drain(ch1, stg_b, sem_b)

      @pl.when(s1 >= 3)
      def _():
        drain_flush(ch1 - 64, rows_vb, semo_b)

      xpose(stg_b, rows_vb)
      flush(ch1, rows_vb, semo_b)

    return carry

  lax.fori_loop(0, (_NCH + 63) // 64 + 1, pair, 0)
  drain_flush(w, rows_va, semo_a)
  drain_flush(w + 32, rows_vb, semo_b)

  @pl.when(w < _NTILE % 8)
  def _():
    g = _NCH * 8 + w
    pltpu.sync_copy(t4_hbm.at[0, pl.ds(g, 1), :, :], stg_a.at[pl.ds(0, 1), :, :])
    pltpu.sync_copy(t4_hbm.at[1, pl.ds(g, 1), :, :], stg_a.at[pl.ds(8, 1), :, :])
    xsel = (iota // 8) * 8

    @plsc.parallel_loop(0, 128, step=1, unroll=16)
    def _(j):
      rows_va[j, :] = plsc.load_gather(stg_a, [xsel, rsel, iota * 0 + j])
    pltpu.sync_copy(rows_va.at[pl.ds(0, 128), :],
                    out_hbm.at[pl.ds(g * 128, 128), :])


@functools.partial(
    pl.kernel,
    mesh=plsc.VectorSubcoreMesh(core_axis_name="c", subcore_axis_name="s"),
    out_type=jax.ShapeDtypeStruct((_B, _DO), jnp.float32),
    compiler_params=pltpu.CompilerParams(use_tc_tiling_on_sc=False),
    scratch_types=[
        pltpu.VMEM((_CB * _SEQ,), jnp.int32),
        pltpu.VMEM((_CB * _SEQ,), jnp.int32),
        pltpu.VMEM((_CB * _SEQ, _DO), jnp.float32),
        pltpu.VMEM((_CB * _SEQ, _DO), jnp.float32),
        pltpu.VMEM((_CB, _DO), jnp.float32),
        pltpu.VMEM((_DO,), jnp.float32),
        pltpu.SemaphoreType.DMA,
        pltpu.SemaphoreType.DMA,
    ],
)
def _bagsum(ids_hbm, t2_hbm, b_hbm, out_hbm, idx_a, idx_b, rows_a, rows_b,
            stg_v, bias_v, sem_a, sem_b):
  w = lax.axis_index("s") * 2 + lax.axis_index("c")
  pltpu.sync_copy(b_hbm, bias_v)
  bias = bias_v[...]

  def fetch(c, idx_v, rows_v, sem):
    pltpu.sync_copy(
        ids_hbm.at[pl.ds(w * _BAGS_W * _SEQ + c * _CB * _SEQ, _CB * _SEQ)],
        idx_v)
    for g in range(_NGRP):
      pltpu.async_copy(t2_hbm.at[idx_v.at[pl.ds(g * _GRP, _GRP)]],
                       rows_v.at[pl.ds(g * _GRP, _GRP), :], sem)

  def drain(idx_v, rows_v, sem):
    for g in range(_NGRP):
      pltpu.make_async_copy(t2_hbm.at[idx_v.at[pl.ds(g * _GRP, _GRP)]],
                            rows_v.at[pl.ds(g * _GRP, _GRP), :], sem).wait()

  def reduce(c, rows_v):
    def red(i, accs):
      return tuple(accs[b] + rows_v[b * _SEQ + i, :] for b in range(_CB))

    accs = lax.fori_loop(0, _SEQ, red, tuple(bias for _ in range(_CB)))
    for b in range(_CB):
      stg_v[b, :] = accs[b]
    pltpu.sync_copy(stg_v, out_hbm.at[pl.ds(w * _BAGS_W + c * _CB, _CB), :])

  fetch(0, idx_a, rows_a, sem_a)

  def pair(t, carry):
    c = 2 * t
    fetch(c + 1, idx_b, rows_b, sem_b)
    drain(idx_a, rows_a, sem_a)
    reduce(c, rows_a)
    fetch(jnp.minimum(c + 2, _CHUNKS - 1), idx_a, rows_a, sem_a)
    drain(idx_b, rows_b, sem_b)
    reduce(c + 1, rows_b)
    return carry

  lax.fori_loop(0, _CHUNKS // 2, pair, 0)
  drain(idx_a, rows_a, sem_a)


def kernel(input_ids, table, fc_w, fc_b):
  t2t = _transform(lax.optimization_barrier(table.T), fc_w)
  t4 = t2t.reshape(2, 8, _NTILE, 128).transpose(0, 2, 1, 3)
  t2 = _untile(t4)
  ids_r = input_ids.astype(jnp.int32).reshape(_B * _SEQ)
  return _bagsum(ids_r, t2, fc_b)
